# Initial kernel scaffold; baseline (speedup 1.0000x reference)
import jax
import jax.numpy as jnp
from jax.experimental import pallas as pl


def kernel(questions_tensor, table):
    raise NotImplementedError("write your pallas kernel here")


# speedup vs baseline: 1.8854x; 1.8854x over previous
import functools

import jax
import jax.numpy as jnp
from jax import lax
from jax.experimental import pallas as pl
from jax.experimental.pallas import tpu as pltpu
from jax.experimental.pallas import tpu_sc as plsc

_NC = 2
_NS = 16
_NW = _NC * _NS

_EMBED = 64
_CH = 128
_NBUF = 8


@functools.lru_cache(maxsize=None)
def _make_gather(vocab, batch_flat):
    assert batch_flat % (_NW * _CH) == 0
    k_steps = batch_flat // (_NW * _CH)
    b_per_w = k_steps * _CH

    mesh = plsc.VectorSubcoreMesh(core_axis_name="c", subcore_axis_name="s")

    scratch = [pltpu.VMEM((k_steps, _CH), jnp.int32)]
    scratch += [pltpu.VMEM((_CH, _EMBED), jnp.float32) for _ in range(_NBUF)]
    scratch += [pltpu.SemaphoreType.DMA for _ in range(_NBUF)]

    @functools.partial(
        pl.kernel,
        mesh=mesh,
        out_type=jax.ShapeDtypeStruct((batch_flat, _EMBED), jnp.float32),
        scratch_types=scratch,
        compiler_params=pltpu.CompilerParams(use_tc_tiling_on_sc=False),
    )
    def k(table_hbm, idx_hbm, out_hbm, idx_v, *bufs_and_sems):
        rows = bufs_and_sems[:_NBUF]
        sems = bufs_and_sems[_NBUF:]
        wid = lax.axis_index("s") * _NC + lax.axis_index("c")
        base = wid * b_per_w

        pltpu.sync_copy(idx_hbm.at[wid], idx_v)

        def fire(g, b):
            pltpu.async_copy(table_hbm.at[idx_v.at[g]], rows[b], sems[b])

        for b in range(_NBUF):
            fire(b, b)

        def outer(i, carry):
            go = i * _NBUF
            for b in range(_NBUF):
                g = go + b
                pltpu.make_async_copy(
                    table_hbm.at[idx_v.at[g]], rows[b], sems[b]).wait()
                pltpu.sync_copy(rows[b],
                                out_hbm.at[pl.ds(base + g * _CH, _CH)])

                @pl.when(g + _NBUF < k_steps)
                def _():
                    fire(g + _NBUF, b)
            return carry

        lax.fori_loop(0, k_steps // _NBUF, outer, 0)

    return k


def kernel(questions_tensor, table):
    batch, seq = questions_tensor.shape
    vocab, embed = table.shape
    flat = batch * seq
    idx = questions_tensor.reshape(_NW, flat // (_NW * _CH), _CH)
    out = _make_gather(vocab, flat)(table, idx)
    return out.reshape(batch, seq, embed)
